# baseline (speedup 1.0000x reference)
import jax
import jax.numpy as jnp
from jax import lax
from jax.experimental import pallas as pl
from jax.experimental.pallas import tpu as pltpu
from jax.experimental.pallas import tpu_sc as plsc

N_ATOMS = 320000
N_IN = 128
N_HIDDEN = 64
N_OUT = 3
NUM_MOL = 2048

_NW = 32
_CHUNK = N_ATOMS // _NW
_ACC = NUM_MOL * N_OUT


_MLP_BLOCK = 32000


def _mlp_body(x_ref, w1_ref, b1_ref, w2t_ref, b2_ref, yt_ref):
    x = x_ref[...]
    h = jnp.dot(x, w1_ref[...], preferred_element_type=jnp.float32)
    h = h + b1_ref[...]
    h = h * (0.5 * jnp.tanh(0.5 * h) + 0.5)
    yt = lax.dot_general(
        w2t_ref[...], h, (((1,), (1,)), ((), ())),
        preferred_element_type=jnp.float32,
    )
    yt_ref[...] = yt + b2_ref[...]


def _run_mlp(x, w1, b1, w2t, b2):
    grid = N_ATOMS // _MLP_BLOCK
    return pl.pallas_call(
        _mlp_body,
        grid=(grid,),
        in_specs=[
            pl.BlockSpec((_MLP_BLOCK, N_IN), lambda i: (i, 0)),
            pl.BlockSpec((N_IN, N_HIDDEN), lambda i: (0, 0)),
            pl.BlockSpec((1, N_HIDDEN), lambda i: (0, 0)),
            pl.BlockSpec((N_OUT, N_HIDDEN), lambda i: (0, 0)),
            pl.BlockSpec((N_OUT, 1), lambda i: (0, 0)),
        ],
        out_specs=pl.BlockSpec((N_OUT, _MLP_BLOCK), lambda i: (0, i)),
        out_shape=jax.ShapeDtypeStruct((N_OUT, N_ATOMS), jnp.float32),
    )(x, w1, b1, w2t, b2)


def _sc_scatter_body(y_hbm, idx_hbm, out_hbm, y_v, idx_v, acc_v, sems):
    cid = lax.axis_index("c")
    sid = lax.axis_index("s")
    wid = sid * 2 + cid
    base = wid * _CHUNK

    cp_idx = pltpu.async_copy(
        idx_hbm.at[pl.ds(base, _CHUNK)], idx_v.at[pl.ds(0, _CHUNK)], sems[0]
    )
    cp_y = []
    for c in range(N_OUT):
        cp_y.append(
            pltpu.async_copy(
                y_hbm.at[pl.ds(c * N_ATOMS + base, _CHUNK)],
                y_v.at[pl.ds(c * _CHUNK, _CHUNK)],
                sems[1 + c],
            )
        )

    zeros = jnp.zeros((16,), jnp.float32)

    def zero_body(j, _):
        acc_v[pl.ds(j * 16, 16)] = zeros
        return 0

    lax.fori_loop(0, _ACC // 16, zero_body, 0, unroll=8)

    cp_idx.wait()
    for cp in cp_y:
        cp.wait()

    iota = lax.iota(jnp.int32, 16)
    last_lane = iota == 15
    not_last = iota != 15

    def body(i, _):
        b = i * 16
        idx16 = idx_v[pl.ds(b, 16)]
        tgt = idx16 * N_OUT
        s3 = []
        for c in range(N_OUT):
            yv = y_v[pl.ds(c * _CHUNK + b, 16)]
            s3.append(plsc.cumsum(yv))
        uniform = idx16[0] == idx16[15]

        @pl.when(uniform)
        def _():
            for c in range(N_OUT):
                plsc.addupdate_scatter(acc_v, [tgt + c], s3[c], mask=last_lane)

        @pl.when(jnp.logical_not(uniform))
        def _():
            idxp1 = idx_v[pl.ds(b + 1, 16)]
            neq = idx16 != idxp1
            m_add = neq | last_lane
            m_sub = neq & not_last
            tgtp1 = idxp1 * N_OUT
            for c in range(N_OUT):
                plsc.addupdate_scatter(acc_v, [tgt + c], s3[c], mask=m_add)
                plsc.addupdate_scatter(acc_v, [tgtp1 + c], -s3[c], mask=m_sub)

        return 0

    lax.fori_loop(0, _CHUNK // 16, body, 0, unroll=8)

    pltpu.sync_copy(acc_v, out_hbm.at[wid])


def _run_sc_scatter(y_flat, idx):
    mesh = plsc.VectorSubcoreMesh(core_axis_name="c", subcore_axis_name="s")
    fn = pl.kernel(
        _sc_scatter_body,
        out_type=jax.ShapeDtypeStruct((_NW, _ACC), jnp.float32),
        mesh=mesh,
        scratch_types=[
            pltpu.VMEM((_CHUNK * N_OUT,), jnp.float32),
            pltpu.VMEM((_CHUNK + 16,), jnp.int32),
            pltpu.VMEM((_ACC,), jnp.float32),
            [pltpu.SemaphoreType.DMA] * 4,
        ],
        compiler_params=pltpu.CompilerParams(needs_layout_passes=False),
    )
    return fn(y_flat, idx)


def _reduce_body(p_ref, o_ref):
    o_ref[...] = jnp.sum(p_ref[...], axis=0, keepdims=True)


def _run_reduce(partials):
    return pl.pallas_call(
        _reduce_body,
        out_shape=jax.ShapeDtypeStruct((1, _ACC), jnp.float32),
    )(partials)


def kernel(scalar_representation, idx_m, W1, b1, W2, b2):
    y = _run_mlp(
        scalar_representation,
        W1,
        b1.reshape(1, N_HIDDEN),
        W2.T,
        b2.reshape(N_OUT, 1),
    )
    partials = _run_sc_scatter(y.reshape(-1), idx_m.astype(jnp.int32))
    out = _run_reduce(partials)
    return out.reshape(NUM_MOL, N_OUT)


# speedup vs baseline: 1.0059x; 1.0059x over previous
import jax
import jax.numpy as jnp
from jax import lax
from jax.experimental import pallas as pl
from jax.experimental.pallas import tpu as pltpu
from jax.experimental.pallas import tpu_sc as plsc

N_ATOMS = 320000
N_IN = 128
N_HIDDEN = 64
N_OUT = 3
NUM_MOL = 2048

_NW = 32
_CHUNK = N_ATOMS // _NW
_ACC = NUM_MOL * N_OUT


_MLP_BLOCK = 32000


def _mlp_body(x_ref, w1_ref, b1_ref, w2t_ref, b2_ref, yt_ref):
    x = x_ref[...]
    h = jnp.dot(x, w1_ref[...], preferred_element_type=jnp.float32)
    h = h + b1_ref[...]
    h = h * (0.5 * jnp.tanh(0.5 * h) + 0.5)
    yt = lax.dot_general(
        w2t_ref[...], h, (((1,), (1,)), ((), ())),
        preferred_element_type=jnp.float32,
    )
    yt_ref[...] = yt + b2_ref[...]


def _run_mlp(x, w1, b1, w2t, b2):
    grid = N_ATOMS // _MLP_BLOCK
    return pl.pallas_call(
        _mlp_body,
        grid=(grid,),
        in_specs=[
            pl.BlockSpec((_MLP_BLOCK, N_IN), lambda i: (i, 0)),
            pl.BlockSpec((N_IN, N_HIDDEN), lambda i: (0, 0)),
            pl.BlockSpec((1, N_HIDDEN), lambda i: (0, 0)),
            pl.BlockSpec((N_OUT, N_HIDDEN), lambda i: (0, 0)),
            pl.BlockSpec((N_OUT, 1), lambda i: (0, 0)),
        ],
        out_specs=pl.BlockSpec((N_OUT, _MLP_BLOCK), lambda i: (0, i)),
        out_shape=jax.ShapeDtypeStruct((N_OUT, N_ATOMS), jnp.float32),
    )(x, w1, b1, w2t, b2)


def _sc_scatter_body(y_hbm, idx_hbm, out_hbm, y_v, idx_v, acc_v, sems):
    cid = lax.axis_index("c")
    sid = lax.axis_index("s")
    wid = sid * 2 + cid
    base = wid * _CHUNK

    cp_idx = pltpu.async_copy(
        idx_hbm.at[pl.ds(base, _CHUNK)], idx_v.at[pl.ds(0, _CHUNK)], sems[0]
    )
    cp_y = []
    for c in range(N_OUT):
        cp_y.append(
            pltpu.async_copy(
                y_hbm.at[pl.ds(c * N_ATOMS + base, _CHUNK)],
                y_v.at[pl.ds(c * _CHUNK, _CHUNK)],
                sems[1 + c],
            )
        )

    zeros = jnp.zeros((16,), jnp.float32)

    def zero_body(j, _):
        acc_v[pl.ds(j * 16, 16)] = zeros
        return 0

    lax.fori_loop(0, _ACC // 16, zero_body, 0, unroll=8)

    cp_idx.wait()
    for cp in cp_y:
        cp.wait()

    iota = lax.iota(jnp.int32, 16)
    last_lane = iota == 15
    not_last = iota != 15

    def body(i, _):
        b = i * 16
        idx16 = idx_v[pl.ds(b, 16)]
        tgt = idx16 * N_OUT
        s3 = []
        for c in range(N_OUT):
            yv = y_v[pl.ds(c * _CHUNK + b, 16)]
            s3.append(plsc.cumsum(yv))
        uniform = idx16[0] == idx16[15]

        @pl.when(uniform)
        def _():
            for c in range(N_OUT):
                plsc.addupdate_scatter(acc_v, [tgt + c], s3[c], mask=last_lane)

        @pl.when(jnp.logical_not(uniform))
        def _():
            idxp1 = idx_v[pl.ds(b + 1, 16)]
            neq = idx16 != idxp1
            m_add = neq | last_lane
            m_sub = neq & not_last
            tgtp1 = idxp1 * N_OUT
            for c in range(N_OUT):
                plsc.addupdate_scatter(acc_v, [tgt + c], s3[c], mask=m_add)
                plsc.addupdate_scatter(acc_v, [tgtp1 + c], -s3[c], mask=m_sub)

        return 0

    lax.fori_loop(0, _CHUNK // 16, body, 0, unroll=4)

    pltpu.sync_copy(acc_v, out_hbm.at[wid])


def _run_sc_scatter(y_flat, idx):
    mesh = plsc.VectorSubcoreMesh(core_axis_name="c", subcore_axis_name="s")
    fn = pl.kernel(
        _sc_scatter_body,
        out_type=jax.ShapeDtypeStruct((_NW, _ACC), jnp.float32),
        mesh=mesh,
        scratch_types=[
            pltpu.VMEM((_CHUNK * N_OUT,), jnp.float32),
            pltpu.VMEM((_CHUNK + 16,), jnp.int32),
            pltpu.VMEM((_ACC,), jnp.float32),
            [pltpu.SemaphoreType.DMA] * 4,
        ],
        compiler_params=pltpu.CompilerParams(needs_layout_passes=False),
    )
    return fn(y_flat, idx)


def _reduce_body(p_ref, o_ref):
    o_ref[...] = jnp.sum(p_ref[...], axis=0, keepdims=True)


def _run_reduce(partials):
    return pl.pallas_call(
        _reduce_body,
        out_shape=jax.ShapeDtypeStruct((1, _ACC), jnp.float32),
    )(partials)


def kernel(scalar_representation, idx_m, W1, b1, W2, b2):
    y = _run_mlp(
        scalar_representation,
        W1,
        b1.reshape(1, N_HIDDEN),
        W2.T,
        b2.reshape(N_OUT, 1),
    )
    partials = _run_sc_scatter(y.reshape(-1), idx_m.astype(jnp.int32))
    out = _run_reduce(partials)
    return out.reshape(NUM_MOL, N_OUT)
